# baseline (speedup 1.0000x reference)
import functools

import jax
import jax.numpy as jnp
from jax import lax
from jax.experimental import pallas as pl
from jax.experimental.pallas import tpu as pltpu
from jax.experimental.pallas import tpu_sc as plsc

N = 10000
NPAD = 10240
E = 160000
E2 = E // 2
H = 256
NUM_MOLS = 256
NC = 2
NS = 16
CH = 128
NCHUNKS = E // CH
HC = H // NC


def _pack2(a, b):
    ua = jax.lax.bitcast_convert_type(a, jnp.uint32)
    ub = jax.lax.bitcast_convert_type(b, jnp.uint32)
    ra = (ua + jnp.uint32(0x7FFF) + ((ua >> 16) & jnp.uint32(1))) >> 16
    rb = (ub + jnp.uint32(0x7FFF) + ((ub >> 16) & jnp.uint32(1))) & jnp.uint32(0xFFFF0000)
    return jax.lax.bitcast_convert_type(ra | rb, jnp.int32)


def _unlo(x):
    u = jax.lax.bitcast_convert_type(x, jnp.uint32)
    return jax.lax.bitcast_convert_type(u << 16, jnp.float32)


def _unhi(x):
    u = jax.lax.bitcast_convert_type(x, jnp.uint32)
    return jax.lax.bitcast_convert_type(u & jnp.uint32(0xFFFF0000), jnp.float32)


def _tab_body(first, final, *refs):
    if first:
        x_ref, w_ref, b_ref = refs[:3]
        outs = refs[3:]
        x = x_ref[...]
    else:
        ts_ref, agga_ref, aggb_ref, w_ref, b_ref = refs[:5]
        outs = refs[5:]
        ag = jnp.concatenate([agga_ref[0] + aggb_ref[0],
                              agga_ref[1] + aggb_ref[1]], axis=1)
        x = jnp.maximum(ts_ref[...] + ag, 0.0)
    y = jnp.dot(x, w_ref[...], preferred_element_type=jnp.float32)
    y = y + b_ref[0:1, :]
    if final:
        v = y[:, :H]
        w = y[:, H:]
        outs[0][...] = _pack2(v[:, :HC], v[:, HC:])
        outs[1][...] = _pack2(w[:, :HC], w[:, HC:])
    else:
        outs[0][...] = _pack2(y[:, :H], y[:, H:2 * H])
        outs[1][...] = _pack2(y[:, 2 * H:2 * H + HC], y[:, 2 * H + HC:3 * H])
        outs[2][...] = y[:, 3 * H:]


def _tables_call(first, final, x_or_ts, agga, aggb, w, b2):
    BN = 1024
    grid = (NPAD // BN,)
    dout = w.shape[1]
    k = x_or_ts.shape[1]
    if final:
        out_shape = [jax.ShapeDtypeStruct((NPAD, HC), jnp.int32),
                     jax.ShapeDtypeStruct((NPAD, HC), jnp.int32)]
        out_specs = [pl.BlockSpec((BN, HC), lambda i: (i, 0)),
                     pl.BlockSpec((BN, HC), lambda i: (i, 0))]
    else:
        out_shape = [jax.ShapeDtypeStruct((NPAD, H), jnp.int32),
                     jax.ShapeDtypeStruct((NPAD, HC), jnp.int32),
                     jax.ShapeDtypeStruct((NPAD, H), jnp.float32)]
        out_specs = [pl.BlockSpec((BN, H), lambda i: (i, 0)),
                     pl.BlockSpec((BN, HC), lambda i: (i, 0)),
                     pl.BlockSpec((BN, H), lambda i: (i, 0))]
    if first:
        in_specs = [pl.BlockSpec((BN, k), lambda i: (i, 0))]
        args = (x_or_ts,)
    else:
        in_specs = [pl.BlockSpec((BN, H), lambda i: (i, 0)),
                    pl.BlockSpec((NC, BN, HC), lambda i: (0, i, 0)),
                    pl.BlockSpec((NC, BN, HC), lambda i: (0, i, 0))]
        args = (x_or_ts, agga, aggb)
    in_specs += [pl.BlockSpec((k, dout), lambda i: (0, 0)),
                 pl.BlockSpec((8, dout), lambda i: (0, 0))]
    body = functools.partial(_tab_body, first, final)
    return pl.pallas_call(body, grid=grid, in_specs=in_specs,
                          out_specs=out_specs, out_shape=out_shape)(*args, w, b2)


def _edges_body(hb_ref, w_ref, b_ref, gvn_ref, gw_ref, nb_ref, msg_ref):
    eh = jnp.dot(hb_ref[...].astype(jnp.bfloat16),
                 w_ref[...].astype(jnp.bfloat16),
                 preferred_element_type=jnp.float32)
    vn = gvn_ref[...]
    wx = jnp.concatenate([_unlo(gw_ref[...]), _unhi(gw_ref[...])], axis=1)
    s = eh + b_ref[0:1, :] + _unlo(vn) + wx
    nb_ref[...] = jnp.maximum(s, 0.0)
    m = jax.nn.sigmoid(s) * _unhi(vn)
    msg_ref[0] = m[:, :HC]
    msg_ref[1] = m[:, HC:]


def _edges_call(hb, w, b2, gvn, gw):
    BE = 1000
    grid = (E2 // BE,)
    k = hb.shape[1]
    out_shape = [jax.ShapeDtypeStruct((E2, H), jnp.float32),
                 jax.ShapeDtypeStruct((NC, E2, HC), jnp.float32)]
    out_specs = [pl.BlockSpec((BE, H), lambda i: (i, 0)),
                 pl.BlockSpec((NC, BE, HC), lambda i: (0, i, 0))]
    in_specs = [pl.BlockSpec((BE, k), lambda i: (i, 0)),
                pl.BlockSpec((k, H), lambda i: (0, 0)),
                pl.BlockSpec((8, H), lambda i: (0, 0)),
                pl.BlockSpec((BE, H), lambda i: (i, 0)),
                pl.BlockSpec((BE, HC), lambda i: (i, 0))]
    return pl.pallas_call(_edges_body, grid=grid, in_specs=in_specs,
                          out_specs=out_specs, out_shape=out_shape)(hb, w, b2, gvn, gw)


def _fedges_body(hb_ref, w_ref, b_ref, gv_ref, gw_ref, ids_ref, out_ref, cnt_ref):
    i = pl.program_id(0)
    y = jnp.dot(hb_ref[...].astype(jnp.bfloat16),
                w_ref[...].astype(jnp.bfloat16),
                preferred_element_type=jnp.float32)
    y = y + b_ref[0:1, :]
    gv = jnp.concatenate([_unlo(gv_ref[...]), _unhi(gv_ref[...])], axis=1)
    gw = jnp.concatenate([_unlo(gw_ref[...]), _unhi(gw_ref[...])], axis=1)
    s = y[:, :H] + gv + gw
    m = jax.nn.sigmoid(s) * y[:, H:]
    out_ref[0] = m[:, :HC]
    out_ref[1] = m[:, HC:]
    be = ids_ref.shape[0]
    oh = (ids_ref[...] == jax.lax.broadcasted_iota(jnp.int32, (be, NUM_MOLS), 1))
    cnt = jnp.dot(oh.astype(jnp.float32).T, jnp.ones((be, 8), jnp.float32),
                  preferred_element_type=jnp.float32)

    @pl.when(i == 0)
    def _():
        cnt_ref[...] = jnp.zeros_like(cnt_ref)

    cnt_ref[...] += cnt


def _fedges_call(hb, w, b2, gv, gw, ids):
    BE = 1000
    grid = (E2 // BE,)
    out_shape = [jax.ShapeDtypeStruct((NC, E2, HC), jnp.float32),
                 jax.ShapeDtypeStruct((NUM_MOLS, 8), jnp.float32)]
    out_specs = [pl.BlockSpec((NC, BE, HC), lambda i: (0, i, 0)),
                 pl.BlockSpec((NUM_MOLS, 8), lambda i: (0, 0))]
    in_specs = [pl.BlockSpec((BE, H), lambda i: (i, 0)),
                pl.BlockSpec((H, 2 * H), lambda i: (0, 0)),
                pl.BlockSpec((8, 2 * H), lambda i: (0, 0)),
                pl.BlockSpec((BE, HC), lambda i: (i, 0)),
                pl.BlockSpec((BE, HC), lambda i: (i, 0)),
                pl.BlockSpec((BE, 1), lambda i: (i, 0))]
    return pl.pallas_call(_fedges_body, grid=grid, in_specs=in_specs,
                          out_specs=out_specs, out_shape=out_shape)(
                              hb, w, b2, gv, gw, ids)


def _div_body(sa_ref, sb_ref, ca_ref, cb_ref, out_ref):
    c = jnp.maximum(ca_ref[:, 0:1] + cb_ref[:, 0:1], 1.0)
    out_ref[:, :HC] = (sa_ref[0] + sb_ref[0]) / c
    out_ref[:, HC:] = (sa_ref[1] + sb_ref[1]) / c


def _div_call(sums3a, sums3b, ca, cb):
    return pl.pallas_call(
        _div_body,
        out_shape=jax.ShapeDtypeStruct((NUM_MOLS, H), jnp.float32),
    )(sums3a, sums3b, ca, cb)


def _sc_gather(t1, t2, idx1, idx2, d1, d2):
    mesh = plsc.VectorSubcoreMesh(core_axis_name="c", subcore_axis_name="s")
    nch = E2 // CH

    @functools.partial(
        pl.kernel, mesh=mesh,
        out_type=[jax.ShapeDtypeStruct((E2, d1), jnp.int32),
                  jax.ShapeDtypeStruct((E2, d2), jnp.int32)],
        scratch_types=[pltpu.VMEM((CH,), jnp.int32),
                       pltpu.VMEM((CH,), jnp.int32),
                       pltpu.VMEM((CH,), jnp.int32),
                       pltpu.VMEM((CH,), jnp.int32),
                       pltpu.VMEM((CH, d1), jnp.int32),
                       pltpu.VMEM((CH, d2), jnp.int32),
                       pltpu.VMEM((CH, d1), jnp.int32),
                       pltpu.VMEM((CH, d2), jnp.int32),
                       pltpu.SemaphoreType.DMA,
                       pltpu.SemaphoreType.DMA,
                       pltpu.SemaphoreType.DMA,
                       pltpu.SemaphoreType.DMA,
                       pltpu.SemaphoreType.DMA],
    )
    def k(t1_hbm, t2_hbm, i1_hbm, i2_hbm, o1_hbm, o2_hbm,
          i1a, i2a, i1b, i2b, b1a, b2a, b1b, b2b, gsem, wsa, wsb, isa, isb):
        cid = lax.axis_index("c")
        sid = lax.axis_index("s")
        wid = sid * NC + cid
        nw = NC * NS
        sets = ((i1a, i2a, b1a, b2a, wsa, isa), (i1b, i2b, b1b, b2b, wsb, isb))

        def loadidx(i, b):
            chunk = wid + i * nw
            i1_v, i2_v, _, _, _, isem = sets[b]

            @pl.when(chunk < nch)
            def _():
                base = chunk * CH
                pltpu.async_copy(i1_hbm.at[pl.ds(base, CH)], i1_v, isem)
                pltpu.async_copy(i2_hbm.at[pl.ds(base, CH)], i2_v, isem)

        def handle(i, b, first_use):
            chunk = wid + i * nw
            i1_v, i2_v, b1_v, b2_v, ws, isem = sets[b]

            @pl.when(chunk < nch)
            def _():
                base = chunk * CH
                pltpu.make_async_copy(i1_hbm.at[pl.ds(base, CH)], i1_v, isem).wait()
                pltpu.make_async_copy(i2_hbm.at[pl.ds(base, CH)], i2_v, isem).wait()
                if not first_use:
                    pltpu.make_async_copy(b1_v, o1_hbm.at[pl.ds(base, CH)], ws).wait()
                    pltpu.make_async_copy(b2_v, o2_hbm.at[pl.ds(base, CH)], ws).wait()
                pltpu.async_copy(t1_hbm.at[i1_v], b1_v, gsem)
                pltpu.async_copy(t2_hbm.at[i2_v], b2_v, gsem)
                pltpu.make_async_copy(t1_hbm.at[i1_v], b1_v, gsem).wait()
                pltpu.make_async_copy(t2_hbm.at[i2_v], b2_v, gsem).wait()
                pltpu.async_copy(b1_v, o1_hbm.at[pl.ds(base, CH)], ws)
                pltpu.async_copy(b2_v, o2_hbm.at[pl.ds(base, CH)], ws)
            loadidx(i + 2, b)

        loadidx(0, 0)
        loadidx(1, 1)
        handle(0, 0, True)
        handle(1, 1, True)

        def body(j, carry):
            handle(2 * j + 2, 0, False)
            handle(2 * j + 3, 1, False)
            return carry

        niter = (nch + nw - 1) // nw
        lax.fori_loop(0, (niter - 2 + 1) // 2, body, 0)

        def drain(b):
            _, _, b1_v, b2_v, ws, _ = sets[b]
            pltpu.make_async_copy(b1_v, o1_hbm.at[pl.ds(0, CH)], ws).wait()
            pltpu.make_async_copy(b2_v, o2_hbm.at[pl.ds(0, CH)], ws).wait()

        drain(0)
        drain(1)

    return k(t1, t2, idx1, idx2)


def _sc_scatter(msg3, dstv):
    mesh = plsc.VectorSubcoreMesh(core_axis_name="c", subcore_axis_name="s")
    rows_per_sub = NPAD // NS
    nch = E2 // CH

    @functools.partial(
        pl.kernel, mesh=mesh,
        out_type=jax.ShapeDtypeStruct((NC, NPAD, HC), jnp.float32),
        scratch_types=[pltpu.VMEM((CH,), jnp.int32),
                       pltpu.VMEM((CH, HC), jnp.float32),
                       pltpu.VMEM((CH,), jnp.int32),
                       pltpu.VMEM((CH, HC), jnp.float32),
                       pltpu.VMEM_SHARED((NPAD, HC), jnp.float32),
                       pltpu.SemaphoreType.DMA,
                       pltpu.SemaphoreType.DMA],
    )
    def k(msg_hbm, dst_hbm, agg_hbm, ia, ba, ib, bb, acc_sh, rsa, rsb):
        cid = lax.axis_index("c")
        sid = lax.axis_index("s")
        sets = ((ia, ba, rsa), (ib, bb, rsb))
        zer = jnp.zeros((16,), jnp.float32)

        def zrow(r, carry):
            for j in range(HC // 16):
                ba[r, j * 16:(j + 1) * 16] = zer
            return carry

        lax.fori_loop(0, CH, zrow, 0)

        def zcp(kk, carry):
            pltpu.sync_copy(ba, acc_sh.at[pl.ds(sid * rows_per_sub + kk * CH, CH)])
            return carry

        lax.fori_loop(0, rows_per_sub // CH, zcp, 0)
        plsc.subcore_barrier()
        niter = (nch + NS - 1) // NS

        def load(i, b):
            chunk = sid + i * NS
            idx_v, buf_v, rs = sets[b]

            @pl.when(chunk < nch)
            def _():
                base = chunk * CH
                pltpu.async_copy(dst_hbm.at[pl.ds(base, CH)], idx_v, rs)
                pltpu.async_copy(msg_hbm.at[cid, pl.ds(base, CH)], buf_v, rs)

        def scat(i, b):
            chunk = sid + i * NS
            idx_v, buf_v, rs = sets[b]

            @pl.when(chunk < nch)
            def _():
                pltpu.make_async_copy(dst_hbm.at[pl.ds(0, CH)], idx_v, rs).wait()
                pltpu.make_async_copy(msg_hbm.at[cid, pl.ds(0, CH)], buf_v, rs).wait()
                pltpu.sync_copy(buf_v, acc_sh.at[idx_v], add=True)

        load(0, 0)
        load(1, 1)

        def body(j, carry):
            i0 = 2 * j
            scat(i0, 0)
            load(i0 + 2, 0)
            scat(i0 + 1, 1)
            load(i0 + 3, 1)
            return carry

        lax.fori_loop(0, niter // 2, body, 0)
        plsc.subcore_barrier()

        def flsh(kk, carry):
            r0 = sid * rows_per_sub + kk * CH
            pltpu.sync_copy(acc_sh.at[pl.ds(r0, CH)], ba)
            pltpu.sync_copy(ba, agg_hbm.at[cid, pl.ds(r0, CH)])
            return carry

        lax.fori_loop(0, rows_per_sub // CH, flsh, 0)

    return k(msg3, dstv)


def _sc_scatter_mols(gated3, ids):
    mesh = plsc.VectorSubcoreMesh(core_axis_name="c", subcore_axis_name="s")
    nch = E2 // CH

    @functools.partial(
        pl.kernel, mesh=mesh,
        out_type=jax.ShapeDtypeStruct((NC, NUM_MOLS, HC), jnp.float32),
        scratch_types=[pltpu.VMEM((CH,), jnp.int32),
                       pltpu.VMEM((CH, HC), jnp.float32),
                       pltpu.VMEM_SHARED((NUM_MOLS, HC), jnp.float32),
                       pltpu.SemaphoreType.DMA],
    )
    def k(g_hbm, ids_hbm, sums_hbm, idx_v, buf_v, acc_sh, sem):
        cid = lax.axis_index("c")
        sid = lax.axis_index("s")
        zer = jnp.zeros((16,), jnp.float32)

        def zrow(r, carry):
            for j in range(HC // 16):
                buf_v[r, j * 16:(j + 1) * 16] = zer
            return carry

        lax.fori_loop(0, CH, zrow, 0)

        @pl.when(sid < NUM_MOLS // CH)
        def _():
            pltpu.sync_copy(buf_v, acc_sh.at[pl.ds(sid * CH, CH)])

        plsc.subcore_barrier()

        def body(i, carry):
            chunk = sid + i * NS

            @pl.when(chunk < nch)
            def _():
                base = chunk * CH
                pltpu.sync_copy(ids_hbm.at[pl.ds(base, CH)], idx_v)
                pltpu.sync_copy(g_hbm.at[cid, pl.ds(base, CH)], buf_v)
                pltpu.sync_copy(buf_v, acc_sh.at[idx_v], add=True)
            return carry

        lax.fori_loop(0, (nch + NS - 1) // NS, body, 0)
        plsc.subcore_barrier()

        @pl.when(sid < NUM_MOLS // CH)
        def _():
            pltpu.sync_copy(acc_sh.at[pl.ds(sid * CH, CH)], buf_v)
            pltpu.sync_copy(buf_v, sums_hbm.at[cid, pl.ds(sid * CH, CH)])

    return k(gated3, ids)


def _b2(b):
    return jnp.tile(b[None, :], (8, 1))


def kernel(atom_features, bond_features, edge_index, bond_mol_ids, params):
    src = (edge_index[0, :E2], edge_index[0, E2:])
    dst = (edge_index[1, :E2], edge_index[1, E2:])
    ids = (bond_mol_ids[:E2], bond_mol_ids[E2:])
    x0 = jnp.pad(atom_features, ((0, NPAD - N), (0, 128 - atom_features.shape[1])))
    bf = jnp.pad(bond_features, ((0, 0), (0, 128 - bond_features.shape[1])))
    hb = (bf[:E2], bf[E2:])
    ts = None
    agg = None
    for li, lp in enumerate(params["layers"]):
        wcat = jnp.concatenate([lp["V"]["w"], lp["W_nei"]["w"],
                                lp["W"]["w"], lp["W_self"]["w"]], axis=1)
        bcat = jnp.concatenate([lp["V"]["b"], lp["W_nei"]["b"],
                                lp["W"]["b"], lp["W_self"]["b"]])
        if li == 0:
            wcat = jnp.pad(wcat, ((0, 128 - wcat.shape[0]), (0, 0)))
            tsn, tw, tself = _tables_call(True, False, x0, None, None,
                                          wcat, _b2(bcat))
        else:
            tsn, tw, tself = _tables_call(False, False, ts, agg[0], agg[1],
                                          wcat, _b2(bcat))
        wb = lp["W_bond"]["w"]
        if li == 0:
            wb = jnp.pad(wb, ((0, 128 - wb.shape[0]), (0, 0)))
        bb2 = _b2(lp["W_bond"]["b"])
        g = [None, None]
        for h in range(2):
            g[h] = _sc_gather(tsn, tw, src[h], dst[h], H, HC)
        nbs = [None, None]
        aggs = [None, None]
        for h in range(2):
            nbs[h], msg3 = _edges_call(hb[h], wb, bb2, g[h][0], g[h][1])
            aggs[h] = _sc_scatter(msg3, dst[h])
        agg = (aggs[0], aggs[1])
        ts = tself
        hb = (nbs[0], nbs[1])
    wvw = jnp.concatenate([params["V"]["w"], params["W"]["w"]], axis=1)
    bvw = jnp.concatenate([params["V"]["b"], params["W"]["b"]])
    tv, tw2 = _tables_call(False, True, ts, agg[0], agg[1], wvw, _b2(bvw))
    wua = jnp.concatenate([params["U"]["w"], params["A"]["w"]], axis=1)
    bua = jnp.concatenate([params["U"]["b"], params["A"]["b"]])
    gf = [None, None]
    for h in range(2):
        gf[h] = _sc_gather(tv, tw2, src[h], dst[h], HC, HC)
    sums = [None, None]
    cnts = [None, None]
    for h in range(2):
        gated3, cnts[h] = _fedges_call(hb[h], wua, _b2(bua),
                                       gf[h][0], gf[h][1], ids[h][:, None])
        sums[h] = _sc_scatter_mols(gated3, ids[h])
    return _div_call(sums[0], sums[1], cnts[0], cnts[1])


# speedup vs baseline: 1.0649x; 1.0649x over previous
import functools

import jax
import jax.numpy as jnp
from jax import lax
from jax.experimental import pallas as pl
from jax.experimental.pallas import tpu as pltpu
from jax.experimental.pallas import tpu_sc as plsc

N = 10000
NPAD = 10240
E = 160000
E2 = E // 2
H = 256
NUM_MOLS = 256
NC = 2
NS = 16
CH = 128
NCHUNKS = E // CH
HC = H // NC


def _pack2(a, b):
    ua = jax.lax.bitcast_convert_type(a, jnp.uint32)
    ub = jax.lax.bitcast_convert_type(b, jnp.uint32)
    ra = (ua + jnp.uint32(0x7FFF) + ((ua >> 16) & jnp.uint32(1))) >> 16
    rb = (ub + jnp.uint32(0x7FFF) + ((ub >> 16) & jnp.uint32(1))) & jnp.uint32(0xFFFF0000)
    return jax.lax.bitcast_convert_type(ra | rb, jnp.int32)


def _unlo(x):
    u = jax.lax.bitcast_convert_type(x, jnp.uint32)
    return jax.lax.bitcast_convert_type(u << 16, jnp.float32)


def _unhi(x):
    u = jax.lax.bitcast_convert_type(x, jnp.uint32)
    return jax.lax.bitcast_convert_type(u & jnp.uint32(0xFFFF0000), jnp.float32)


def _tab_body(first, final, *refs):
    if first:
        x_ref, w_ref, b_ref = refs[:3]
        outs = refs[3:]
        x = x_ref[...]
    else:
        ts_ref, agga_ref, aggb_ref, w_ref, b_ref = refs[:5]
        outs = refs[5:]
        ag = jnp.concatenate([agga_ref[0] + aggb_ref[0],
                              agga_ref[1] + aggb_ref[1]], axis=1)
        x = jnp.maximum(ts_ref[...] + ag, 0.0)
    y = jnp.dot(x, w_ref[...], preferred_element_type=jnp.float32)
    y = y + b_ref[0:1, :]
    if final:
        v = y[:, :H]
        w = y[:, H:]
        outs[0][...] = _pack2(v[:, :HC], v[:, HC:])
        outs[1][...] = _pack2(w[:, :HC], w[:, HC:])
    else:
        outs[0][...] = _pack2(y[:, :H], y[:, H:2 * H])
        outs[1][...] = _pack2(y[:, 2 * H:2 * H + HC], y[:, 2 * H + HC:3 * H])
        outs[2][...] = y[:, 3 * H:]


def _tables_call(first, final, x_or_ts, agga, aggb, w, b2):
    BN = 1024
    grid = (NPAD // BN,)
    dout = w.shape[1]
    k = x_or_ts.shape[1]
    if final:
        out_shape = [jax.ShapeDtypeStruct((NPAD, HC), jnp.int32),
                     jax.ShapeDtypeStruct((NPAD, HC), jnp.int32)]
        out_specs = [pl.BlockSpec((BN, HC), lambda i: (i, 0)),
                     pl.BlockSpec((BN, HC), lambda i: (i, 0))]
    else:
        out_shape = [jax.ShapeDtypeStruct((NPAD, H), jnp.int32),
                     jax.ShapeDtypeStruct((NPAD, HC), jnp.int32),
                     jax.ShapeDtypeStruct((NPAD, H), jnp.float32)]
        out_specs = [pl.BlockSpec((BN, H), lambda i: (i, 0)),
                     pl.BlockSpec((BN, HC), lambda i: (i, 0)),
                     pl.BlockSpec((BN, H), lambda i: (i, 0))]
    if first:
        in_specs = [pl.BlockSpec((BN, k), lambda i: (i, 0))]
        args = (x_or_ts,)
    else:
        in_specs = [pl.BlockSpec((BN, H), lambda i: (i, 0)),
                    pl.BlockSpec((NC, BN, HC), lambda i: (0, i, 0)),
                    pl.BlockSpec((NC, BN, HC), lambda i: (0, i, 0))]
        args = (x_or_ts, agga, aggb)
    in_specs += [pl.BlockSpec((k, dout), lambda i: (0, 0)),
                 pl.BlockSpec((8, dout), lambda i: (0, 0))]
    body = functools.partial(_tab_body, first, final)
    return pl.pallas_call(body, grid=grid, in_specs=in_specs,
                          out_specs=out_specs, out_shape=out_shape)(*args, w, b2)


def _edges_body(hb_ref, w_ref, b_ref, gvn_ref, gw_ref, nb_ref, msg_ref):
    eh = jnp.dot(hb_ref[...], w_ref[...], preferred_element_type=jnp.float32)
    vn = gvn_ref[...]
    wx = jnp.concatenate([_unlo(gw_ref[...]), _unhi(gw_ref[...])], axis=1)
    s = eh + b_ref[0:1, :] + _unlo(vn) + wx
    nb_ref[...] = jnp.maximum(s, 0.0)
    m = jax.nn.sigmoid(s) * _unhi(vn)
    msg_ref[0] = m[:, :HC]
    msg_ref[1] = m[:, HC:]


def _edges_call(hb, w, b2, gvn, gw):
    BE = 1000
    grid = (E2 // BE,)
    k = hb.shape[1]
    out_shape = [jax.ShapeDtypeStruct((E2, H), jnp.float32),
                 jax.ShapeDtypeStruct((NC, E2, HC), jnp.float32)]
    out_specs = [pl.BlockSpec((BE, H), lambda i: (i, 0)),
                 pl.BlockSpec((NC, BE, HC), lambda i: (0, i, 0))]
    in_specs = [pl.BlockSpec((BE, k), lambda i: (i, 0)),
                pl.BlockSpec((k, H), lambda i: (0, 0)),
                pl.BlockSpec((8, H), lambda i: (0, 0)),
                pl.BlockSpec((BE, H), lambda i: (i, 0)),
                pl.BlockSpec((BE, HC), lambda i: (i, 0))]
    return pl.pallas_call(_edges_body, grid=grid, in_specs=in_specs,
                          out_specs=out_specs, out_shape=out_shape)(hb, w, b2, gvn, gw)


def _fedges_body(hb_ref, w_ref, b_ref, gv_ref, gw_ref, ids_ref,
                 sums_ref, cnt_ref):
    i = pl.program_id(0)
    y = jnp.dot(hb_ref[...].astype(jnp.bfloat16),
                w_ref[...].astype(jnp.bfloat16),
                preferred_element_type=jnp.float32)
    y = y + b_ref[0:1, :]
    gv = jnp.concatenate([_unlo(gv_ref[...]), _unhi(gv_ref[...])], axis=1)
    gw = jnp.concatenate([_unlo(gw_ref[...]), _unhi(gw_ref[...])], axis=1)
    s = y[:, :H] + gv + gw
    m = jax.nn.sigmoid(s) * y[:, H:]
    be = ids_ref.shape[0]
    oh = (ids_ref[...] == jax.lax.broadcasted_iota(jnp.int32, (be, NUM_MOLS), 1))
    oht = oh.astype(jnp.bfloat16).T
    sums = jnp.dot(oht, m.astype(jnp.bfloat16),
                   preferred_element_type=jnp.float32)
    cnt = jnp.dot(oht, jnp.ones((be, 8), jnp.bfloat16),
                  preferred_element_type=jnp.float32)

    @pl.when(i == 0)
    def _():
        sums_ref[...] = jnp.zeros_like(sums_ref)
        cnt_ref[...] = jnp.zeros_like(cnt_ref)

    sums_ref[...] += sums
    cnt_ref[...] += cnt


def _fedges_call(hb, w, b2, gv, gw, ids):
    BE = 1000
    grid = (E2 // BE,)
    out_shape = [jax.ShapeDtypeStruct((NUM_MOLS, H), jnp.float32),
                 jax.ShapeDtypeStruct((NUM_MOLS, 8), jnp.float32)]
    out_specs = [pl.BlockSpec((NUM_MOLS, H), lambda i: (0, 0)),
                 pl.BlockSpec((NUM_MOLS, 8), lambda i: (0, 0))]
    in_specs = [pl.BlockSpec((BE, H), lambda i: (i, 0)),
                pl.BlockSpec((H, 2 * H), lambda i: (0, 0)),
                pl.BlockSpec((8, 2 * H), lambda i: (0, 0)),
                pl.BlockSpec((BE, HC), lambda i: (i, 0)),
                pl.BlockSpec((BE, HC), lambda i: (i, 0)),
                pl.BlockSpec((BE, 1), lambda i: (i, 0))]
    return pl.pallas_call(_fedges_body, grid=grid, in_specs=in_specs,
                          out_specs=out_specs, out_shape=out_shape)(
                              hb, w, b2, gv, gw, ids)


def _div_body(sa_ref, sb_ref, ca_ref, cb_ref, out_ref):
    c = jnp.maximum(ca_ref[:, 0:1] + cb_ref[:, 0:1], 1.0)
    out_ref[...] = (sa_ref[...] + sb_ref[...]) / c


def _div_call(sa, sb, ca, cb):
    return pl.pallas_call(
        _div_body,
        out_shape=jax.ShapeDtypeStruct((NUM_MOLS, H), jnp.float32),
    )(sa, sb, ca, cb)


def _sc_gather(t1, t2, idx1, idx2, d1, d2):
    mesh = plsc.VectorSubcoreMesh(core_axis_name="c", subcore_axis_name="s")
    nch = E2 // CH

    @functools.partial(
        pl.kernel, mesh=mesh,
        out_type=[jax.ShapeDtypeStruct((E2, d1), jnp.int32),
                  jax.ShapeDtypeStruct((E2, d2), jnp.int32)],
        scratch_types=[pltpu.VMEM((CH,), jnp.int32),
                       pltpu.VMEM((CH,), jnp.int32),
                       pltpu.VMEM((CH,), jnp.int32),
                       pltpu.VMEM((CH,), jnp.int32),
                       pltpu.VMEM((CH, d1), jnp.int32),
                       pltpu.VMEM((CH, d2), jnp.int32),
                       pltpu.VMEM((CH, d1), jnp.int32),
                       pltpu.VMEM((CH, d2), jnp.int32),
                       pltpu.SemaphoreType.DMA,
                       pltpu.SemaphoreType.DMA,
                       pltpu.SemaphoreType.DMA,
                       pltpu.SemaphoreType.DMA,
                       pltpu.SemaphoreType.DMA],
    )
    def k(t1_hbm, t2_hbm, i1_hbm, i2_hbm, o1_hbm, o2_hbm,
          i1a, i2a, i1b, i2b, b1a, b2a, b1b, b2b, gsem, wsa, wsb, isa, isb):
        cid = lax.axis_index("c")
        sid = lax.axis_index("s")
        wid = sid * NC + cid
        nw = NC * NS
        sets = ((i1a, i2a, b1a, b2a, wsa, isa), (i1b, i2b, b1b, b2b, wsb, isb))

        def loadidx(i, b):
            chunk = wid + i * nw
            i1_v, i2_v, _, _, _, isem = sets[b]

            @pl.when(chunk < nch)
            def _():
                base = chunk * CH
                pltpu.async_copy(i1_hbm.at[pl.ds(base, CH)], i1_v, isem)
                pltpu.async_copy(i2_hbm.at[pl.ds(base, CH)], i2_v, isem)

        def handle(i, b, first_use):
            chunk = wid + i * nw
            i1_v, i2_v, b1_v, b2_v, ws, isem = sets[b]

            @pl.when(chunk < nch)
            def _():
                base = chunk * CH
                pltpu.make_async_copy(i1_hbm.at[pl.ds(base, CH)], i1_v, isem).wait()
                pltpu.make_async_copy(i2_hbm.at[pl.ds(base, CH)], i2_v, isem).wait()
                if not first_use:
                    pltpu.make_async_copy(b1_v, o1_hbm.at[pl.ds(base, CH)], ws).wait()
                    pltpu.make_async_copy(b2_v, o2_hbm.at[pl.ds(base, CH)], ws).wait()
                pltpu.async_copy(t1_hbm.at[i1_v], b1_v, gsem)
                pltpu.async_copy(t2_hbm.at[i2_v], b2_v, gsem)
                pltpu.make_async_copy(t1_hbm.at[i1_v], b1_v, gsem).wait()
                pltpu.make_async_copy(t2_hbm.at[i2_v], b2_v, gsem).wait()
                pltpu.async_copy(b1_v, o1_hbm.at[pl.ds(base, CH)], ws)
                pltpu.async_copy(b2_v, o2_hbm.at[pl.ds(base, CH)], ws)
            loadidx(i + 2, b)

        loadidx(0, 0)
        loadidx(1, 1)
        handle(0, 0, True)
        handle(1, 1, True)

        def body(j, carry):
            handle(2 * j + 2, 0, False)
            handle(2 * j + 3, 1, False)
            return carry

        niter = (nch + nw - 1) // nw
        lax.fori_loop(0, (niter - 2 + 1) // 2, body, 0)

        def drain(b):
            _, _, b1_v, b2_v, ws, _ = sets[b]
            pltpu.make_async_copy(b1_v, o1_hbm.at[pl.ds(0, CH)], ws).wait()
            pltpu.make_async_copy(b2_v, o2_hbm.at[pl.ds(0, CH)], ws).wait()

        drain(0)
        drain(1)

    return k(t1, t2, idx1, idx2)


def _sc_scatter(msg3, dstv):
    mesh = plsc.VectorSubcoreMesh(core_axis_name="c", subcore_axis_name="s")
    rows_per_sub = NPAD // NS
    nch = E2 // CH

    @functools.partial(
        pl.kernel, mesh=mesh,
        out_type=jax.ShapeDtypeStruct((NC, NPAD, HC), jnp.float32),
        scratch_types=[pltpu.VMEM((CH,), jnp.int32),
                       pltpu.VMEM((CH, HC), jnp.float32),
                       pltpu.VMEM((CH,), jnp.int32),
                       pltpu.VMEM((CH, HC), jnp.float32),
                       pltpu.VMEM_SHARED((NPAD, HC), jnp.float32),
                       pltpu.SemaphoreType.DMA,
                       pltpu.SemaphoreType.DMA],
    )
    def k(msg_hbm, dst_hbm, agg_hbm, ia, ba, ib, bb, acc_sh, rsa, rsb):
        cid = lax.axis_index("c")
        sid = lax.axis_index("s")
        sets = ((ia, ba, rsa), (ib, bb, rsb))
        zer = jnp.zeros((16,), jnp.float32)

        def zrow(r, carry):
            for j in range(HC // 16):
                ba[r, j * 16:(j + 1) * 16] = zer
            return carry

        lax.fori_loop(0, CH, zrow, 0)

        def zcp(kk, carry):
            pltpu.sync_copy(ba, acc_sh.at[pl.ds(sid * rows_per_sub + kk * CH, CH)])
            return carry

        lax.fori_loop(0, rows_per_sub // CH, zcp, 0)
        plsc.subcore_barrier()
        niter = (nch + NS - 1) // NS

        def load(i, b):
            chunk = sid + i * NS
            idx_v, buf_v, rs = sets[b]

            @pl.when(chunk < nch)
            def _():
                base = chunk * CH
                pltpu.async_copy(dst_hbm.at[pl.ds(base, CH)], idx_v, rs)
                pltpu.async_copy(msg_hbm.at[cid, pl.ds(base, CH)], buf_v, rs)

        def scat(i, b):
            chunk = sid + i * NS
            idx_v, buf_v, rs = sets[b]

            @pl.when(chunk < nch)
            def _():
                pltpu.make_async_copy(dst_hbm.at[pl.ds(0, CH)], idx_v, rs).wait()
                pltpu.make_async_copy(msg_hbm.at[cid, pl.ds(0, CH)], buf_v, rs).wait()
                pltpu.sync_copy(buf_v, acc_sh.at[idx_v], add=True)

        load(0, 0)
        load(1, 1)

        def body(j, carry):
            i0 = 2 * j
            scat(i0, 0)
            load(i0 + 2, 0)
            scat(i0 + 1, 1)
            load(i0 + 3, 1)
            return carry

        lax.fori_loop(0, niter // 2, body, 0)
        plsc.subcore_barrier()

        def flsh(kk, carry):
            r0 = sid * rows_per_sub + kk * CH
            pltpu.sync_copy(acc_sh.at[pl.ds(r0, CH)], ba)
            pltpu.sync_copy(ba, agg_hbm.at[cid, pl.ds(r0, CH)])
            return carry

        lax.fori_loop(0, rows_per_sub // CH, flsh, 0)

    return k(msg3, dstv)


def _b2(b):
    return jnp.tile(b[None, :], (8, 1))


def kernel(atom_features, bond_features, edge_index, bond_mol_ids, params):
    src = (edge_index[0, :E2], edge_index[0, E2:])
    dst = (edge_index[1, :E2], edge_index[1, E2:])
    ids = (bond_mol_ids[:E2], bond_mol_ids[E2:])
    x0 = jnp.pad(atom_features, ((0, NPAD - N), (0, 128 - atom_features.shape[1])))
    bf = jnp.pad(bond_features, ((0, 0), (0, 128 - bond_features.shape[1])))
    hb = (bf[:E2], bf[E2:])
    ts = None
    agg = None
    for li, lp in enumerate(params["layers"]):
        wcat = jnp.concatenate([lp["V"]["w"], lp["W_nei"]["w"],
                                lp["W"]["w"], lp["W_self"]["w"]], axis=1)
        bcat = jnp.concatenate([lp["V"]["b"], lp["W_nei"]["b"],
                                lp["W"]["b"], lp["W_self"]["b"]])
        if li == 0:
            wcat = jnp.pad(wcat, ((0, 128 - wcat.shape[0]), (0, 0)))
            tsn, tw, tself = _tables_call(True, False, x0, None, None,
                                          wcat, _b2(bcat))
        else:
            tsn, tw, tself = _tables_call(False, False, ts, agg[0], agg[1],
                                          wcat, _b2(bcat))
        wb = lp["W_bond"]["w"]
        if li == 0:
            wb = jnp.pad(wb, ((0, 128 - wb.shape[0]), (0, 0)))
        bb2 = _b2(lp["W_bond"]["b"])
        g = [None, None]
        for h in range(2):
            g[h] = _sc_gather(tsn, tw, src[h], dst[h], H, HC)
        nbs = [None, None]
        aggs = [None, None]
        for h in range(2):
            nbs[h], msg3 = _edges_call(hb[h], wb, bb2, g[h][0], g[h][1])
            aggs[h] = _sc_scatter(msg3, dst[h])
        agg = (aggs[0], aggs[1])
        ts = tself
        hb = (nbs[0], nbs[1])
    wvw = jnp.concatenate([params["V"]["w"], params["W"]["w"]], axis=1)
    bvw = jnp.concatenate([params["V"]["b"], params["W"]["b"]])
    tv, tw2 = _tables_call(False, True, ts, agg[0], agg[1], wvw, _b2(bvw))
    wua = jnp.concatenate([params["U"]["w"], params["A"]["w"]], axis=1)
    bua = jnp.concatenate([params["U"]["b"], params["A"]["b"]])
    gf = [None, None]
    for h in range(2):
        gf[h] = _sc_gather(tv, tw2, src[h], dst[h], HC, HC)
    sums = [None, None]
    cnts = [None, None]
    for h in range(2):
        sums[h], cnts[h] = _fedges_call(hb[h], wua, _b2(bua),
                                        gf[h][0], gf[h][1], ids[h][:, None])
    return _div_call(sums[0], sums[1], cnts[0], cnts[1])
